# baseline (speedup 1.0000x reference)
import functools

import jax
import jax.numpy as jnp
from jax import lax
from jax.experimental import pallas as pl
from jax.experimental.pallas import tpu as pltpu
from jax.experimental.pallas import tpu_sc as plsc

N_NODES = 10000
N_EDGES = 160000
HID = 64
K_CHEB = 5

NC = 2
NS = 16
LANES = 16

EPT = N_EDGES // NS
CH = 80
NCHUNK = EPT // CH
RPT = N_NODES // NS
SUB = 125
NSUB = RPT // SUB
FG = HID // LANES


TMAX = 18


def _make_prop_kernel():
    mesh = plsc.VectorSubcoreMesh(core_axis_name="c", subcore_axis_name="s")

    def body(t0, nth, row3, col3, nrm3, out5, row_v, col_v, nrm_v,
             rows_v, rows_b, zbuf, bufA, bufB, nth_s, accum, semA, semB,
             semC, semD):
        c = lax.axis_index("c")
        s = lax.axis_index("s")
        pltpu.sync_copy(nth, nth_s)
        nt_half = nth_s[...][0]
        pltpu.sync_copy(row3.at[s], row_v)
        pltpu.sync_copy(col3.at[s], col_v)
        pltpu.sync_copy(nrm3.at[s], nrm_v)
        zero = jnp.zeros((LANES,), jnp.float32)
        for r in range(SUB):
            for f in range(FG):
                zbuf[r, pl.ds(f * LANES, LANES)] = zero

        def k_step(k, t):
            for sub in range(NSUB):
                pltpu.sync_copy(
                    zbuf, accum.at[pl.ds(s * RPT + sub * SUB, SUB)])
            plsc.subcore_barrier()
            km1 = k - 1

            src = out5.at[km1, t]

            def scale(j, buf):
                for g in range(CH // LANES):
                    nrm16 = nrm_v[j, pl.ds(g * LANES, LANES)]
                    for i in range(LANES):
                        e = g * LANES + i
                        sp = nrm16.at[jnp.full((LANES,), i, jnp.int32)].get(
                            mode='promise_in_bounds')
                        for f in range(FG):
                            sl = pl.ds(f * LANES, LANES)
                            buf[e, sl] = buf[e, sl] * sp

            def pair(it, carry):
                j0 = 2 * it
                j1 = j0 + 1
                ga = pltpu.async_copy(src.at[row_v.at[j0]], rows_v, semA)
                gb = pltpu.async_copy(src.at[row_v.at[j1]], rows_b, semB)
                ga.wait()
                scale(j0, rows_v)
                sa = pltpu.async_copy(rows_v, accum.at[col_v.at[j0]], semC,
                                      add=True)
                gb.wait()
                scale(j1, rows_b)
                sb = pltpu.async_copy(rows_b, accum.at[col_v.at[j1]], semD,
                                      add=True)
                sa.wait()
                sb.wait()
                return carry

            lax.fori_loop(0, NCHUNK // 2, pair, jnp.int32(0))
            jlast = NCHUNK - 1
            pltpu.async_copy(src.at[row_v.at[jlast]], rows_v, semA).wait()
            scale(jlast, rows_v)
            pltpu.async_copy(rows_v, accum.at[col_v.at[jlast]], semC,
                             add=True).wait()
            plsc.subcore_barrier()
            alpha = jnp.where(k == 1, 1.0, 2.0).astype(jnp.float32)
            beta = jnp.where(k == 1, 0.0, 1.0).astype(jnp.float32)
            av = jnp.full((LANES,), alpha)
            bv = jnp.full((LANES,), beta)
            km2 = jnp.maximum(k - 2, 0)
            def comb(sub, carry):
                base = s * RPT + sub * SUB
                pltpu.sync_copy(accum.at[pl.ds(base, SUB)], bufA)
                pltpu.sync_copy(out5.at[km2, t, pl.ds(base, SUB)], bufB)
                for r in range(SUB):
                    for f in range(FG):
                        sl = pl.ds(f * LANES, LANES)
                        bufA[r, sl] = av * bufA[r, sl] - bv * bufB[r, sl]
                pltpu.sync_copy(bufA, out5.at[k, t, pl.ds(base, SUB)])
                return carry

            lax.fori_loop(0, NSUB, comb, jnp.int32(0))
            plsc.subcore_barrier()

        def t_step(tl, carry):
            t = c * nt_half + tl
            pltpu.sync_copy(t0.at[t, pl.ds(s * RPT, RPT)],
                            out5.at[0, t, pl.ds(s * RPT, RPT)])
            plsc.subcore_barrier()

            def k_loop(k, carry2):
                k_step(k, t)
                return carry2

            lax.fori_loop(1, K_CHEB, k_loop, jnp.int32(0))
            return carry

        lax.fori_loop(0, nt_half, t_step, jnp.int32(0))

    return pl.kernel(
        body,
        out_type=jax.ShapeDtypeStruct((K_CHEB, TMAX, N_NODES, HID),
                                      jnp.float32),
        mesh=mesh,
        compiler_params=pltpu.CompilerParams(use_tc_tiling_on_sc=False),
        scratch_types=[
            pltpu.VMEM((NCHUNK, CH), jnp.int32),
            pltpu.VMEM((NCHUNK, CH), jnp.int32),
            pltpu.VMEM((NCHUNK, CH), jnp.float32),
            pltpu.VMEM((CH, HID), jnp.float32),
            pltpu.VMEM((CH, HID), jnp.float32),
            pltpu.VMEM((SUB, HID), jnp.float32),
            pltpu.VMEM((SUB, HID), jnp.float32),
            pltpu.VMEM((SUB, HID), jnp.float32),
            pltpu.VMEM((LANES,), jnp.int32),
            pltpu.VMEM_SHARED((N_NODES, HID), jnp.float32),
            pltpu.SemaphoreType.DMA,
            pltpu.SemaphoreType.DMA,
            pltpu.SemaphoreType.DMA,
            pltpu.SemaphoreType.DMA,
        ],
    )


NBLK = 1000
NGRID = N_NODES // NBLK
_PREC = jax.lax.Precision.HIGHEST


def _xla_temporal_conv(X, convs):
    Xp = jnp.transpose(X, (2, 1, 0))[None]
    dn = jax.lax.conv_dimension_numbers(Xp.shape, convs[0][0].shape,
                                        ('NCHW', 'OIHW', 'NCHW'))

    def c(i):
        w, b = convs[i]
        y = jax.lax.conv_general_dilated(Xp, w, (1, 1), 'VALID',
                                         dimension_numbers=dn)
        return y + b[None, :, None, None]

    P = c(0)
    Q = c(1)
    R = c(2)
    H = jax.nn.relu(P * jax.nn.sigmoid(Q) + R)
    return jnp.transpose(H[0], (2, 1, 0))


def _stack_conv_w(convs):
    ws, bs = [], []
    for w, b in convs:
        ws.append(jnp.transpose(w[:, :, 0, :], (2, 1, 0)))
        bs.append(b)
    return jnp.concatenate(ws, axis=2), jnp.concatenate(bs)


def _tconv_body(x0, x1, x2, w, b, o):
    cout3 = w.shape[2]
    cout = cout3 // 3
    acc = jax.lax.dot_general(x0[0], w[0], (((1,), (0,)), ((), ())),
                              precision=_PREC,
                              preferred_element_type=jnp.float32)
    acc += jax.lax.dot_general(x1[0], w[1], (((1,), (0,)), ((), ())),
                               precision=_PREC,
                               preferred_element_type=jnp.float32)
    acc += jax.lax.dot_general(x2[0], w[2], (((1,), (0,)), ((), ())),
                               precision=_PREC,
                               preferred_element_type=jnp.float32)
    acc += b[0]
    P = acc[:, :cout]
    Q = acc[:, cout:2 * cout]
    R = acc[:, 2 * cout:]
    o[0] = jax.nn.relu(P * jax.nn.sigmoid(Q) + R)


def _temporal_conv(X, convs):
    T = X.shape[0]
    cin = X.shape[2]
    w, b = _stack_conv_w(convs)
    cout3 = int(w.shape[2])
    cout = cout3 // 3
    bs2 = b.reshape(1, cout3)
    xspec = lambda dt: pl.BlockSpec((1, NBLK, cin),
                                    lambda t, n, dt=dt: (t + dt, n, 0))
    return pl.pallas_call(
        _tconv_body,
        grid=(T - 2, NGRID),
        in_specs=[xspec(0), xspec(1), xspec(2),
                  pl.BlockSpec((3, cin, cout3), lambda t, n: (0, 0, 0)),
                  pl.BlockSpec((1, cout3), lambda t, n: (0, 0))],
        out_specs=pl.BlockSpec((1, NBLK, cout), lambda t, n: (t, n, 0)),
        out_shape=jax.ShapeDtypeStruct((T - 2, N_NODES, cout), jnp.float32),
    )(X, X, X, w, bs2)


def _cheb_mm_body(st, w, b, o):
    acc = b[0]
    for k in range(K_CHEB):
        acc = acc + jax.lax.dot_general(
            st[k, 0], w[k], (((1,), (0,)), ((), ())), precision=_PREC,
            preferred_element_type=jnp.float32)
    o[0] = jax.nn.relu(acc)


def _cheb_matmul(St, W, b):
    T1 = St.shape[1]
    bs2 = b.reshape(1, HID)
    return pl.pallas_call(
        _cheb_mm_body,
        grid=(T1, NGRID),
        in_specs=[pl.BlockSpec((K_CHEB, 1, NBLK, HID),
                               lambda t, n: (0, t, n, 0)),
                  pl.BlockSpec((K_CHEB, HID, HID), lambda t, n: (0, 0, 0)),
                  pl.BlockSpec((1, HID), lambda t, n: (0, 0))],
        out_specs=pl.BlockSpec((1, NBLK, HID), lambda t, n: (t, n, 0)),
        out_shape=jax.ShapeDtypeStruct((T1, N_NODES, HID), jnp.float32),
    )(St, W, bs2)


def _bn_body(x, gamma, beta, o, *, final_relu):
    xb = x[...]
    cnt = xb.shape[0] * xb.shape[2]
    mean = jnp.sum(xb, axis=(0, 2), keepdims=True) / cnt
    d = xb - mean
    var = jnp.sum(d * d, axis=(0, 2), keepdims=True) / cnt
    inv = jax.lax.rsqrt(var + 1e-5)
    y = d * inv * gamma[0, 0][None, :, None] + beta[0, 0][None, :, None]
    if final_relu:
        y = jax.nn.relu(y)
    o[...] = y


def _batchnorm(T2, gamma, beta, final_relu):
    T2n, _, C = T2.shape
    g2 = gamma.reshape(NGRID, 1, NBLK)
    b2 = beta.reshape(NGRID, 1, NBLK)
    return pl.pallas_call(
        functools.partial(_bn_body, final_relu=final_relu),
        grid=(NGRID,),
        in_specs=[pl.BlockSpec((T2n, NBLK, C), lambda n: (0, n, 0)),
                  pl.BlockSpec((1, 1, NBLK), lambda n: (n, 0, 0)),
                  pl.BlockSpec((1, 1, NBLK), lambda n: (n, 0, 0))],
        out_specs=pl.BlockSpec((T2n, NBLK, C), lambda n: (0, n, 0)),
        out_shape=jax.ShapeDtypeStruct((T2n, N_NODES, C), jnp.float32),
    )(T2, g2, b2)


def kernel(x, edge_index, edge_weight, params):
    edge_weight = jnp.clip(edge_weight, 1e-6, None)
    row = edge_index[0]
    col = edge_index[1]
    w = jnp.where(row == col, 0.0, edge_weight)
    deg = jax.ops.segment_sum(w, row, num_segments=N_NODES)
    dis = jnp.where(deg > 0, jax.lax.rsqrt(jnp.where(deg > 0, deg, 1.0)), 0.0)
    norm = -dis[row] * w * dis[col]

    row3 = row.reshape(NS, NCHUNK, CH)
    col3 = col.reshape(NS, NCHUNK, CH)
    nrm3 = norm.reshape(NS, NCHUNK, CH)

    prop = _make_prop_kernel()
    h = x[0]
    for i in range(4):
        p = params[i]
        T0 = _xla_temporal_conv(h, p['tc1'])
        T1n = T0.shape[0]
        nth = jnp.full((LANES,), T1n // 2, jnp.int32)
        T0p = jnp.pad(T0, ((0, TMAX - T1n), (0, 0), (0, 0)))
        St = prop(T0p, nth, row3, col3, nrm3)[:, :T1n]
        g = jax.nn.relu(
            jnp.einsum('ktnh,kgh->tng', St, p['cheb_W']) + p['cheb_b'])
        T2 = _xla_temporal_conv(g, p['tc2'])
        h = _batchnorm(T2, p['bn_gamma'], p['bn_beta'], final_relu=(i < 3))
    return h[-1][None]


# speedup vs baseline: 1.0032x; 1.0032x over previous
import functools

import jax
import jax.numpy as jnp
from jax import lax
from jax.experimental import pallas as pl
from jax.experimental.pallas import tpu as pltpu
from jax.experimental.pallas import tpu_sc as plsc

N_NODES = 10000
N_EDGES = 160000
HID = 64
K_CHEB = 5

NC = 2
NS = 16
LANES = 16

EPT = N_EDGES // NS
CH = 80
NCHUNK = EPT // CH
RPT = N_NODES // NS
SUB = 125
NSUB = RPT // SUB
FG = HID // LANES


def _make_prop_kernel(T1):
    assert T1 % 2 == 0
    nt_half = T1 // 2
    mesh = plsc.VectorSubcoreMesh(core_axis_name="c", subcore_axis_name="s")

    def body(t0, row3, col3, nrm3, out5, row_v, col_v, nrm_v,
             rows_v, rows_b, rows_c, rows_d, zbuf, bufA, bufB, accum,
             semA, semB, semC, semD, semE, semF, semG, semH):
        c = lax.axis_index("c")
        s = lax.axis_index("s")
        gsems = (semA, semB, semE, semF)
        ssems = (semC, semD, semG, semH)
        pltpu.sync_copy(row3.at[s], row_v)
        pltpu.sync_copy(col3.at[s], col_v)
        pltpu.sync_copy(nrm3.at[s], nrm_v)
        zero = jnp.zeros((LANES,), jnp.float32)
        for r in range(SUB):
            for f in range(FG):
                zbuf[r, pl.ds(f * LANES, LANES)] = zero

        def k_step(k, t):
            for sub in range(NSUB):
                pltpu.sync_copy(
                    zbuf, accum.at[pl.ds(s * RPT + sub * SUB, SUB)])
            plsc.subcore_barrier()
            km1 = k - 1

            src = out5.at[km1, t]

            def scale(j, buf):
                for g in range(CH // LANES):
                    nrm16 = nrm_v[j, pl.ds(g * LANES, LANES)]
                    for i in range(LANES):
                        e = g * LANES + i
                        sp = nrm16.at[jnp.full((LANES,), i, jnp.int32)].get(
                            mode='promise_in_bounds')
                        for f in range(FG):
                            sl = pl.ds(f * LANES, LANES)
                            buf[e, sl] = buf[e, sl] * sp

            def quad(it, carry):
                j0 = 4 * it
                gs = []
                for q, buf in enumerate((rows_v, rows_b, rows_c, rows_d)):
                    gs.append(pltpu.async_copy(src.at[row_v.at[j0 + q]],
                                               buf, gsems[q]))
                ss = []
                for q, buf in enumerate((rows_v, rows_b, rows_c, rows_d)):
                    gs[q].wait()
                    scale(j0 + q, buf)
                    ss.append(pltpu.async_copy(
                        buf, accum.at[col_v.at[j0 + q]], ssems[q],
                        add=True))
                for h in ss:
                    h.wait()
                return carry

            lax.fori_loop(0, NCHUNK // 4, quad, jnp.int32(0))
            jlast = NCHUNK - 1
            pltpu.async_copy(src.at[row_v.at[jlast]], rows_v, semA).wait()
            scale(jlast, rows_v)
            pltpu.async_copy(rows_v, accum.at[col_v.at[jlast]], semC,
                             add=True).wait()
            plsc.subcore_barrier()
            alpha = jnp.where(k == 1, 1.0, 2.0).astype(jnp.float32)
            beta = jnp.where(k == 1, 0.0, 1.0).astype(jnp.float32)
            av = jnp.full((LANES,), alpha)
            bv = jnp.full((LANES,), beta)
            km2 = jnp.maximum(k - 2, 0)
            def comb(sub, carry):
                sb = s * RPT + sub * SUB
                pltpu.sync_copy(accum.at[pl.ds(sb, SUB)], bufA)
                pltpu.sync_copy(out5.at[km2, t, pl.ds(sb, SUB)], bufB)
                for r in range(SUB):
                    for f in range(FG):
                        sl = pl.ds(f * LANES, LANES)
                        bufA[r, sl] = av * bufA[r, sl] - bv * bufB[r, sl]
                pltpu.sync_copy(bufA, out5.at[k, t, pl.ds(sb, SUB)])
                return carry

            lax.fori_loop(0, NSUB, comb, jnp.int32(0))
            plsc.subcore_barrier()

        def t_step(tl, carry):
            t = c * nt_half + tl
            pltpu.sync_copy(t0.at[t, pl.ds(s * RPT, RPT)],
                            out5.at[0, t, pl.ds(s * RPT, RPT)])
            plsc.subcore_barrier()

            def k_loop(k, carry2):
                k_step(k, t)
                return carry2

            lax.fori_loop(1, K_CHEB, k_loop, jnp.int32(0))
            return carry

        lax.fori_loop(0, nt_half, t_step, jnp.int32(0))

    return pl.kernel(
        body,
        out_type=jax.ShapeDtypeStruct((K_CHEB, T1, N_NODES, HID),
                                      jnp.float32),
        mesh=mesh,
        compiler_params=pltpu.CompilerParams(use_tc_tiling_on_sc=False),
        scratch_types=[
            pltpu.VMEM((NCHUNK, CH), jnp.int32),
            pltpu.VMEM((NCHUNK, CH), jnp.int32),
            pltpu.VMEM((NCHUNK, CH), jnp.float32),
            pltpu.VMEM((CH, HID), jnp.float32),
            pltpu.VMEM((CH, HID), jnp.float32),
            pltpu.VMEM((CH, HID), jnp.float32),
            pltpu.VMEM((CH, HID), jnp.float32),
            pltpu.VMEM((SUB, HID), jnp.float32),
            pltpu.VMEM((SUB, HID), jnp.float32),
            pltpu.VMEM((SUB, HID), jnp.float32),
            pltpu.VMEM_SHARED((N_NODES, HID), jnp.float32),
            pltpu.SemaphoreType.DMA,
            pltpu.SemaphoreType.DMA,
            pltpu.SemaphoreType.DMA,
            pltpu.SemaphoreType.DMA,
            pltpu.SemaphoreType.DMA,
            pltpu.SemaphoreType.DMA,
            pltpu.SemaphoreType.DMA,
            pltpu.SemaphoreType.DMA,
        ],
    )


NBLK = 1000
NGRID = N_NODES // NBLK
_PREC = jax.lax.Precision.HIGHEST


def _xla_temporal_conv(X, convs):
    Xp = jnp.transpose(X, (2, 1, 0))[None]
    dn = jax.lax.conv_dimension_numbers(Xp.shape, convs[0][0].shape,
                                        ('NCHW', 'OIHW', 'NCHW'))

    def c(i):
        w, b = convs[i]
        y = jax.lax.conv_general_dilated(Xp, w, (1, 1), 'VALID',
                                         dimension_numbers=dn)
        return y + b[None, :, None, None]

    P = c(0)
    Q = c(1)
    R = c(2)
    H = jax.nn.relu(P * jax.nn.sigmoid(Q) + R)
    return jnp.transpose(H[0], (2, 1, 0))


def _stack_conv_w(convs):
    ws, bs = [], []
    for w, b in convs:
        ws.append(jnp.transpose(w[:, :, 0, :], (2, 1, 0)))
        bs.append(b)
    return jnp.concatenate(ws, axis=2), jnp.concatenate(bs)


def _tconv_body(x0, x1, x2, w, b, o):
    cout3 = w.shape[2]
    cout = cout3 // 3
    acc = jax.lax.dot_general(x0[0], w[0], (((1,), (0,)), ((), ())),
                              precision=_PREC,
                              preferred_element_type=jnp.float32)
    acc += jax.lax.dot_general(x1[0], w[1], (((1,), (0,)), ((), ())),
                               precision=_PREC,
                               preferred_element_type=jnp.float32)
    acc += jax.lax.dot_general(x2[0], w[2], (((1,), (0,)), ((), ())),
                               precision=_PREC,
                               preferred_element_type=jnp.float32)
    acc += b[0]
    P = acc[:, :cout]
    Q = acc[:, cout:2 * cout]
    R = acc[:, 2 * cout:]
    o[0] = jax.nn.relu(P * jax.nn.sigmoid(Q) + R)


def _temporal_conv(X, convs):
    T = X.shape[0]
    cin = X.shape[2]
    w, b = _stack_conv_w(convs)
    cout3 = int(w.shape[2])
    cout = cout3 // 3
    bs2 = b.reshape(1, cout3)
    xspec = lambda dt: pl.BlockSpec((1, NBLK, cin),
                                    lambda t, n, dt=dt: (t + dt, n, 0))
    return pl.pallas_call(
        _tconv_body,
        grid=(T - 2, NGRID),
        in_specs=[xspec(0), xspec(1), xspec(2),
                  pl.BlockSpec((3, cin, cout3), lambda t, n: (0, 0, 0)),
                  pl.BlockSpec((1, cout3), lambda t, n: (0, 0))],
        out_specs=pl.BlockSpec((1, NBLK, cout), lambda t, n: (t, n, 0)),
        out_shape=jax.ShapeDtypeStruct((T - 2, N_NODES, cout), jnp.float32),
    )(X, X, X, w, bs2)


def _cheb_mm_body(st, w, b, o):
    acc = b[0]
    for k in range(K_CHEB):
        acc = acc + jax.lax.dot_general(
            st[k, 0], w[k], (((1,), (0,)), ((), ())), precision=_PREC,
            preferred_element_type=jnp.float32)
    o[0] = jax.nn.relu(acc)


def _cheb_matmul(St, W, b):
    T1 = St.shape[1]
    bs2 = b.reshape(1, HID)
    return pl.pallas_call(
        _cheb_mm_body,
        grid=(T1, NGRID),
        in_specs=[pl.BlockSpec((K_CHEB, 1, NBLK, HID),
                               lambda t, n: (0, t, n, 0)),
                  pl.BlockSpec((K_CHEB, HID, HID), lambda t, n: (0, 0, 0)),
                  pl.BlockSpec((1, HID), lambda t, n: (0, 0))],
        out_specs=pl.BlockSpec((1, NBLK, HID), lambda t, n: (t, n, 0)),
        out_shape=jax.ShapeDtypeStruct((T1, N_NODES, HID), jnp.float32),
    )(St, W, bs2)


def _bn_body(x, gamma, beta, o, *, final_relu):
    xb = x[...]
    cnt = xb.shape[0] * xb.shape[2]
    mean = jnp.sum(xb, axis=(0, 2), keepdims=True) / cnt
    d = xb - mean
    var = jnp.sum(d * d, axis=(0, 2), keepdims=True) / cnt
    inv = jax.lax.rsqrt(var + 1e-5)
    y = d * inv * gamma[0, 0][None, :, None] + beta[0, 0][None, :, None]
    if final_relu:
        y = jax.nn.relu(y)
    o[...] = y


def _batchnorm(T2, gamma, beta, final_relu):
    T2n, _, C = T2.shape
    g2 = gamma.reshape(NGRID, 1, NBLK)
    b2 = beta.reshape(NGRID, 1, NBLK)
    return pl.pallas_call(
        functools.partial(_bn_body, final_relu=final_relu),
        grid=(NGRID,),
        in_specs=[pl.BlockSpec((T2n, NBLK, C), lambda n: (0, n, 0)),
                  pl.BlockSpec((1, 1, NBLK), lambda n: (n, 0, 0)),
                  pl.BlockSpec((1, 1, NBLK), lambda n: (n, 0, 0))],
        out_specs=pl.BlockSpec((T2n, NBLK, C), lambda n: (0, n, 0)),
        out_shape=jax.ShapeDtypeStruct((T2n, N_NODES, C), jnp.float32),
    )(T2, g2, b2)


def kernel(x, edge_index, edge_weight, params):
    edge_weight = jnp.clip(edge_weight, 1e-6, None)
    row = edge_index[0]
    col = edge_index[1]
    w = jnp.where(row == col, 0.0, edge_weight)
    deg = jax.ops.segment_sum(w, row, num_segments=N_NODES)
    dis = jnp.where(deg > 0, jax.lax.rsqrt(jnp.where(deg > 0, deg, 1.0)), 0.0)
    norm = -dis[row] * w * dis[col]

    row3 = row.reshape(NS, NCHUNK, CH)
    col3 = col.reshape(NS, NCHUNK, CH)
    nrm3 = norm.reshape(NS, NCHUNK, CH)

    h = x[0]
    for i in range(4):
        p = params[i]
        T0 = _xla_temporal_conv(h, p['tc1'])
        T1n = T0.shape[0]
        St = _make_prop_kernel(T1n)(T0, row3, col3, nrm3)
        g = jax.nn.relu(
            jnp.einsum('ktnh,kgh->tng', St, p['cheb_W']) + p['cheb_b'])
        T2 = _xla_temporal_conv(g, p['tc2'])
        h = _batchnorm(T2, p['bn_gamma'], p['bn_beta'], final_relu=(i < 3))
    return h[-1][None]


# speedup vs baseline: 1.0308x; 1.0275x over previous
import functools

import jax
import jax.numpy as jnp
from jax import lax
from jax.experimental import pallas as pl
from jax.experimental.pallas import tpu as pltpu
from jax.experimental.pallas import tpu_sc as plsc

N_NODES = 10000
N_EDGES = 160000
HID = 64
K_CHEB = 5

NC = 2
NS = 16
LANES = 16

EPT = N_EDGES // NS
CH = 80
NCHUNK = EPT // CH
RPT = N_NODES // NS
SUB = 125
NSUB = RPT // SUB
FG = HID // LANES


def _make_prop_kernel(T1):
    assert T1 % 2 == 0
    nt_half = T1 // 2
    mesh = plsc.VectorSubcoreMesh(core_axis_name="c", subcore_axis_name="s")

    def body(t0, row3, col3, nrm3, out5, row_v, col_v, nrm_v,
             rows_v, rows_b, zbuf, bufA, bufB, accum,
             semA, semB, semC, semD):
        c = lax.axis_index("c")
        s = lax.axis_index("s")
        pltpu.sync_copy(row3.at[s], row_v)
        pltpu.sync_copy(col3.at[s], col_v)
        pltpu.sync_copy(nrm3.at[s], nrm_v)
        zero = jnp.zeros((LANES,), jnp.float32)
        for r in range(SUB):
            for f in range(FG):
                zbuf[r, pl.ds(f * LANES, LANES)] = zero

        def k_step(k, t):
            for sub in range(NSUB):
                pltpu.sync_copy(
                    zbuf, accum.at[pl.ds(s * RPT + sub * SUB, SUB)])
            plsc.subcore_barrier()
            km1 = k - 1

            src = out5.at[km1, t]

            def scale(j, buf):
                for g in range(CH // LANES):
                    nrm16 = nrm_v[j, pl.ds(g * LANES, LANES)]
                    for i in range(LANES):
                        e = g * LANES + i
                        sp = nrm16.at[jnp.full((LANES,), i, jnp.int32)].get(
                            mode='promise_in_bounds')
                        for f in range(FG):
                            sl = pl.ds(f * LANES, LANES)
                            buf[e, sl] = buf[e, sl] * sp

            def pair(it, carry):
                j0 = 2 * it
                j1 = j0 + 1
                ga = pltpu.async_copy(src.at[row_v.at[j0]], rows_v, semA)
                gb = pltpu.async_copy(src.at[row_v.at[j1]], rows_b, semB)
                ga.wait()
                scale(j0, rows_v)
                sa = pltpu.async_copy(rows_v, accum.at[col_v.at[j0]], semC,
                                      add=True)
                gb.wait()
                scale(j1, rows_b)
                sb = pltpu.async_copy(rows_b, accum.at[col_v.at[j1]], semD,
                                      add=True)
                sa.wait()
                sb.wait()
                return carry

            lax.fori_loop(0, NCHUNK // 2, pair, jnp.int32(0))
            jlast = NCHUNK - 1
            pltpu.async_copy(src.at[row_v.at[jlast]], rows_v, semA).wait()
            scale(jlast, rows_v)
            pltpu.async_copy(rows_v, accum.at[col_v.at[jlast]], semC,
                             add=True).wait()
            plsc.subcore_barrier()
            alpha = jnp.where(k == 1, 1.0, 2.0).astype(jnp.float32)
            beta = jnp.where(k == 1, 0.0, 1.0).astype(jnp.float32)
            av = jnp.full((LANES,), alpha)
            bv = jnp.full((LANES,), beta)
            km2 = jnp.maximum(k - 2, 0)
            def comb(sub, carry):
                sb = s * RPT + sub * SUB
                pltpu.sync_copy(accum.at[pl.ds(sb, SUB)], bufA)
                pltpu.sync_copy(out5.at[km2, t, pl.ds(sb, SUB)], bufB)
                for r in range(SUB):
                    for f in range(FG):
                        sl = pl.ds(f * LANES, LANES)
                        bufA[r, sl] = av * bufA[r, sl] - bv * bufB[r, sl]
                pltpu.sync_copy(bufA, out5.at[k, t, pl.ds(sb, SUB)])
                return carry

            lax.fori_loop(0, NSUB, comb, jnp.int32(0))
            plsc.subcore_barrier()

        def t_step(tl, carry):
            t = c * nt_half + tl
            pltpu.sync_copy(t0.at[t, pl.ds(s * RPT, RPT)],
                            out5.at[0, t, pl.ds(s * RPT, RPT)])
            plsc.subcore_barrier()

            def k_loop(k, carry2):
                k_step(k, t)
                return carry2

            lax.fori_loop(1, K_CHEB, k_loop, jnp.int32(0))
            return carry

        lax.fori_loop(0, nt_half, t_step, jnp.int32(0))

    return pl.kernel(
        body,
        out_type=jax.ShapeDtypeStruct((K_CHEB, T1, N_NODES, HID),
                                      jnp.float32),
        mesh=mesh,
        compiler_params=pltpu.CompilerParams(use_tc_tiling_on_sc=False),
        scratch_types=[
            pltpu.VMEM((NCHUNK, CH), jnp.int32),
            pltpu.VMEM((NCHUNK, CH), jnp.int32),
            pltpu.VMEM((NCHUNK, CH), jnp.float32),
            pltpu.VMEM((CH, HID), jnp.float32),
            pltpu.VMEM((CH, HID), jnp.float32),
            pltpu.VMEM((SUB, HID), jnp.float32),
            pltpu.VMEM((SUB, HID), jnp.float32),
            pltpu.VMEM((SUB, HID), jnp.float32),
            pltpu.VMEM_SHARED((N_NODES, HID), jnp.float32),
            pltpu.SemaphoreType.DMA,
            pltpu.SemaphoreType.DMA,
            pltpu.SemaphoreType.DMA,
            pltpu.SemaphoreType.DMA,
        ],
    )


NBLK = 1000
NGRID = N_NODES // NBLK


def _xla_temporal_conv(X, convs):
    Xp = jnp.transpose(X, (2, 1, 0))[None]
    dn = jax.lax.conv_dimension_numbers(Xp.shape, convs[0][0].shape,
                                        ('NCHW', 'OIHW', 'NCHW'))

    def c(i):
        w, b = convs[i]
        y = jax.lax.conv_general_dilated(Xp, w, (1, 1), 'VALID',
                                         dimension_numbers=dn)
        return y + b[None, :, None, None]

    P = c(0)
    Q = c(1)
    R = c(2)
    H = jax.nn.relu(P * jax.nn.sigmoid(Q) + R)
    return jnp.transpose(H[0], (2, 1, 0))


def _bn_body(x, gamma, beta, o, *, final_relu):
    xb = x[...]
    cnt = xb.shape[0] * xb.shape[2]
    mean = jnp.sum(xb, axis=(0, 2), keepdims=True) / cnt
    d = xb - mean
    var = jnp.sum(d * d, axis=(0, 2), keepdims=True) / cnt
    inv = jax.lax.rsqrt(var + 1e-5)
    y = d * inv * gamma[0, 0][None, :, None] + beta[0, 0][None, :, None]
    if final_relu:
        y = jax.nn.relu(y)
    o[...] = y


def _batchnorm(T2, gamma, beta, final_relu):
    T2n, _, C = T2.shape
    g2 = gamma.reshape(NGRID, 1, NBLK)
    b2 = beta.reshape(NGRID, 1, NBLK)
    return pl.pallas_call(
        functools.partial(_bn_body, final_relu=final_relu),
        grid=(NGRID,),
        in_specs=[pl.BlockSpec((T2n, NBLK, C), lambda n: (0, n, 0)),
                  pl.BlockSpec((1, 1, NBLK), lambda n: (n, 0, 0)),
                  pl.BlockSpec((1, 1, NBLK), lambda n: (n, 0, 0))],
        out_specs=pl.BlockSpec((T2n, NBLK, C), lambda n: (0, n, 0)),
        out_shape=jax.ShapeDtypeStruct((T2n, N_NODES, C), jnp.float32),
    )(T2, g2, b2)


def kernel(x, edge_index, edge_weight, params):
    edge_weight = jnp.clip(edge_weight, 1e-6, None)
    row = edge_index[0]
    col = edge_index[1]
    w = jnp.where(row == col, 0.0, edge_weight)
    deg = jax.ops.segment_sum(w, row, num_segments=N_NODES)
    dis = jnp.where(deg > 0, jax.lax.rsqrt(jnp.where(deg > 0, deg, 1.0)), 0.0)
    norm = -dis[row] * w * dis[col]

    row3 = row.reshape(NS, NCHUNK, CH)
    col3 = col.reshape(NS, NCHUNK, CH)
    nrm3 = norm.reshape(NS, NCHUNK, CH)

    h = x[0]
    for i in range(4):
        p = params[i]
        T0 = _xla_temporal_conv(h, p['tc1'])
        T1n = T0.shape[0]
        St = _make_prop_kernel(T1n)(T0, row3, col3, nrm3)
        g = jax.nn.relu(
            jnp.einsum('ktnh,kgh->tng', St, p['cheb_W']) + p['cheb_b'])
        T2 = _xla_temporal_conv(g, p['tc2'])
        h = _batchnorm(T2, p['bn_gamma'], p['bn_beta'], final_relu=(i < 3))
    return h[-1][None]
